# Initial kernel scaffold; baseline (speedup 1.0000x reference)
import jax
import jax.numpy as jnp
from jax.experimental import pallas as pl


def kernel(K_value, index, normed_A_value, A_index, x, n1, n2, W1, b1, W2, b2, W3, b3, W4, b4):
    raise NotImplementedError("write your pallas kernel here")


# speedup vs baseline: 4.1258x; 4.1258x over previous
import functools

import jax
import jax.numpy as jnp
from jax import lax
from jax.experimental import pallas as pl
from jax.experimental.pallas import tpu as pltpu
from jax.experimental.pallas import tpu_sc as plsc

_N = 10000
_E = 320000
_D = 128
_NC = 2
_NS = 16
_NW = _NC * _NS
_EPW = _E // _NW
_CHUNK = 80
_NCHUNK = _EPW // _CHUNK
_NP = 10240
_RPT = _NP // _NS
_ZR = 128


def _mlp_body(x_ref, w1, b1, w2, b2, w3, b3, w4, b4, x1_ref, m2_ref):
    xb = x_ref[...]
    dn = (((1,), (1,)), ((), ()))
    h = jnp.maximum(
        lax.dot_general(xb, w1[...], dn, preferred_element_type=jnp.float32)
        + b1[...], 0.0)
    x1_ref[...] = jnp.maximum(
        lax.dot_general(h, w2[...], dn, preferred_element_type=jnp.float32)
        + b2[...], 0.0)
    g = jnp.maximum(
        lax.dot_general(xb, w3[...], dn, preferred_element_type=jnp.float32)
        + b3[...], 0.0)
    m2_ref[...] = jnp.maximum(
        lax.dot_general(g, w4[...], dn, preferred_element_type=jnp.float32)
        + b4[...], 0.0)


def _mlps(x2d, W1, b1, W2, b2, W3, b3, W4, b4):
    BM = 2000
    wspec = pl.BlockSpec((_D, _D), lambda i: (0, 0))
    bspec = pl.BlockSpec((1, _D), lambda i: (0, 0))
    rspec = pl.BlockSpec((BM, _D), lambda i: (i, 0))
    return pl.pallas_call(
        _mlp_body,
        grid=(_N // BM,),
        in_specs=[rspec, wspec, bspec, wspec, bspec, wspec, bspec, wspec, bspec],
        out_specs=[rspec, rspec],
        out_shape=[jax.ShapeDtypeStruct((_N, _D), jnp.float32)] * 2,
    )(x2d, W1, b1.reshape(1, _D), W2, b2.reshape(1, _D),
      W3, b3.reshape(1, _D), W4, b4.reshape(1, _D))


def _add2_body(a_ref, b_ref, o_ref):
    o_ref[...] = a_ref[...] + b_ref[...]


def _add3_body(a_ref, b_ref, c_ref, o_ref):
    o_ref[...] = a_ref[...] + b_ref[...] + c_ref[...]


def _combine(parts, extra=None):
    BM = 2000
    rspec = pl.BlockSpec((BM, _D), lambda i: (i, 0))
    args = [parts[0], parts[1]] + ([] if extra is None else [extra])
    body = _add2_body if extra is None else _add3_body
    return pl.pallas_call(
        body,
        grid=(_N // BM,),
        in_specs=[rspec] * len(args),
        out_specs=rspec,
        out_shape=jax.ShapeDtypeStruct((_N, _D), jnp.float32),
    )(*args)


def _spmm_body(dst_hbm, src_hbm, val_hbm, x_hbm, out_hbm,
               src_v, dst_v, val_v, rows_v, zero_v, acc_sh, sem):
    c = lax.axis_index("c")
    s = lax.axis_index("s")

    zvec = jnp.zeros((16,), jnp.float32)

    def zbody(i, carry):
        zero_v[i // 8, pl.ds((i % 8) * 16, 16)] = zvec
        return carry

    lax.fori_loop(0, _ZR * 8, zbody, 0)
    r0 = s * _RPT
    for t in range(_RPT // _ZR):
        pltpu.sync_copy(zero_v, acc_sh.at[pl.ds(r0 + t * _ZR, _ZR)])
    plsc.subcore_barrier()

    base = (c * _NS + s) * _EPW

    def chunk(j, carry):
        e0 = base + j * _CHUNK
        pltpu.sync_copy(src_hbm.at[pl.ds(e0, _CHUNK)], src_v)
        pltpu.sync_copy(dst_hbm.at[pl.ds(e0, _CHUNK)], dst_v)
        pltpu.sync_copy(val_hbm.at[pl.ds(e0, _CHUNK)], val_v)
        pltpu.async_copy(x_hbm.at[src_v], rows_v, sem).wait()

        def egroup(g, ecarry):
            vv = val_v[pl.ds(g * 16, 16)]
            for l in range(16):
                v = vv[l]
                e = g * 16 + l
                for k in range(_D // 16):
                    rows_v[e, pl.ds(k * 16, 16)] = (
                        rows_v[e, pl.ds(k * 16, 16)] * v)
            return ecarry

        lax.fori_loop(0, _CHUNK // 16, egroup, 0)
        pltpu.sync_copy(rows_v, acc_sh.at[dst_v], add=True)
        return carry

    lax.fori_loop(0, _NCHUNK, chunk, 0)
    plsc.subcore_barrier()
    pltpu.sync_copy(acc_sh.at[pl.ds(r0, _RPT)],
                    out_hbm.at[pl.ds(c * _NP + r0, _RPT)])


def _spmm_partials(dst, src, val, x_mat):
    mesh = plsc.VectorSubcoreMesh(
        core_axis_name="c", subcore_axis_name="s",
        num_cores=_NC, num_subcores=_NS)
    kern = pl.kernel(
        _spmm_body,
        out_type=jax.ShapeDtypeStruct((_NC * _NP, _D), jnp.float32),
        mesh=mesh,
        scratch_types=[
            pltpu.VMEM((_CHUNK,), jnp.int32),
            pltpu.VMEM((_CHUNK,), jnp.int32),
            pltpu.VMEM((_CHUNK,), jnp.float32),
            pltpu.VMEM((_CHUNK, _D), jnp.float32),
            pltpu.VMEM((_ZR, _D), jnp.float32),
            pltpu.VMEM_SHARED((_NP, _D), jnp.float32),
            pltpu.SemaphoreType.DMA,
        ],
    )
    return kern(dst, src, val, x_mat)


def kernel(K_value, index, normed_A_value, A_index, x, n1, n2,
           W1, b1, W2, b2, W3, b3, W4, b4):
    x2d = x.reshape(_N, _D)
    x1, m2 = _mlps(x2d, W1, b1, W2, b2, W3, b3, W4, b4)

    p = _spmm_partials(index[0], index[1], K_value, x1)
    wx = _combine((p[:_N], p[_NP:_NP + _N]))

    q = _spmm_partials(A_index[0], A_index[1], normed_A_value, wx)
    out = _combine((q[:_N], q[_NP:_NP + _N]), extra=m2)
    return out[None]


# speedup vs baseline: 8.5109x; 2.0629x over previous
import jax
import jax.numpy as jnp
from jax import lax
from jax.experimental import pallas as pl
from jax.experimental.pallas import tpu as pltpu
from jax.experimental.pallas import tpu_sc as plsc

_N = 10000
_E = 320000
_D = 128
_NC = 2
_NS = 16
_NW = _NC * _NS
_CHUNK = 64
_NCHUNK = 162
_EPW = _NCHUNK * _CHUNK
_EP = _NW * _EPW
_NP = 10240
_RPT = _NP // _NS


def _mlp_body(x_ref, w1, b1, w2, b2, w3, b3, w4, b4, x1_ref, m2_ref):
    xb = x_ref[...]
    dn = (((1,), (1,)), ((), ()))
    h = jnp.maximum(
        lax.dot_general(xb, w1[...], dn, preferred_element_type=jnp.float32)
        + b1[...], 0.0)
    x1_ref[...] = jnp.maximum(
        lax.dot_general(h, w2[...], dn, preferred_element_type=jnp.float32)
        + b2[...], 0.0)
    g = jnp.maximum(
        lax.dot_general(xb, w3[...], dn, preferred_element_type=jnp.float32)
        + b3[...], 0.0)
    m2_ref[...] = jnp.maximum(
        lax.dot_general(g, w4[...], dn, preferred_element_type=jnp.float32)
        + b4[...], 0.0)


def _mlps(x2d, W1, b1, W2, b2, W3, b3, W4, b4):
    BM = 2000
    wspec = pl.BlockSpec((_D, _D), lambda i: (0, 0))
    bspec = pl.BlockSpec((1, _D), lambda i: (0, 0))
    rspec = pl.BlockSpec((BM, _D), lambda i: (i, 0))
    return pl.pallas_call(
        _mlp_body,
        grid=(_N // BM,),
        in_specs=[rspec, wspec, bspec, wspec, bspec, wspec, bspec, wspec, bspec],
        out_specs=[rspec, rspec],
        out_shape=[jax.ShapeDtypeStruct((_N, _D), jnp.float32)] * 2,
    )(x2d, W1, b1.reshape(1, _D), W2, b2.reshape(1, _D),
      W3, b3.reshape(1, _D), W4, b4.reshape(1, _D))


def _add2_body(a_ref, b_ref, o_ref):
    o_ref[...] = a_ref[...] + b_ref[...]


def _add3_body(a_ref, b_ref, c_ref, o_ref):
    o_ref[...] = a_ref[...] + b_ref[...] + c_ref[...]


def _combine(parts, extra=None):
    BM = 2000
    rspec = pl.BlockSpec((BM, _D), lambda i: (i, 0))
    args = [parts[0], parts[1]] + ([] if extra is None else [extra])
    body = _add2_body if extra is None else _add3_body
    return pl.pallas_call(
        body,
        grid=(_N // BM,),
        in_specs=[rspec] * len(args),
        out_specs=rspec,
        out_shape=jax.ShapeDtypeStruct((_N, _D), jnp.float32),
    )(*args)


def _spmm_body(src_hbm, dst_hbm, val_hbm, x_hbm, out_hbm,
               src_all, dv, vals, rows, acc_sh, gsem, ssem, dvs):
    c = lax.axis_index("c")
    s = lax.axis_index("s")
    w = c * _NS + s

    pltpu.sync_copy(src_hbm.at[w], src_all)

    zvec = jnp.zeros((16,), jnp.float32)

    def zbody(i, carry):
        rows[0][i // 8, pl.ds((i % 8) * 16, 16)] = zvec
        return carry

    lax.fori_loop(0, _CHUNK * 8, zbody, 0)
    r0 = s * _RPT
    for t in range(_RPT // _CHUNK):
        pltpu.sync_copy(rows[0], acc_sh.at[pl.ds(r0 + t * _CHUNK, _CHUNK)])
    plsc.subcore_barrier()

    def gather(j, b):
        pltpu.async_copy(x_hbm.at[src_all.at[j]], rows[b], gsem[b])

    def wait_gather(j, b):
        pltpu.make_async_copy(
            x_hbm.at[src_all.at[j]], rows[b], gsem[b]).wait()

    def dv_load(j, b):
        pltpu.async_copy(dst_hbm.at[w].at[j], dv[b], dvs[b])
        pltpu.async_copy(val_hbm.at[w].at[j], vals[b], dvs[b])

    def wait_dv(j, b):
        pltpu.make_async_copy(dst_hbm.at[w].at[j], dv[b], dvs[b]).wait()
        pltpu.make_async_copy(val_hbm.at[w].at[j], vals[b], dvs[b]).wait()

    def scatter(b):
        pltpu.async_copy(rows[b], acc_sh.at[dv[b].at[0]], ssem[b], add=True)

    def wait_scatter(b):
        pltpu.make_async_copy(
            rows[b], acc_sh.at[dv[b].at[0]], ssem[b]).wait()

    def scale(b):
        vq = vals[b]
        r = rows[b]
        for g in range(_CHUNK // 16):
            vv = vq[pl.ds(g * 16, 16)]
            for l in range(16):
                e = g * 16 + l
                v = vv[l]
                for k in range(_D // 16):
                    r[e, pl.ds(k * 16, 16)] = r[e, pl.ds(k * 16, 16)] * v

    dv_load(0, 0)
    gather(0, 0)
    nq = _NCHUNK // 3

    def body(t, carry):
        for q in range(3):
            j = 3 * t + q
            b = q
            nb = (q + 1) % 3

            if q == 2:
                wait_scatter(nb)
            else:
                @pl.when(t > 0)
                def _():
                    wait_scatter(nb)

            if q == 2:
                @pl.when(t < nq - 1)
                def _():
                    dv_load(j + 1, nb)
                    gather(j + 1, nb)
            else:
                dv_load(j + 1, nb)
                gather(j + 1, nb)

            wait_dv(j, b)
            wait_gather(j, b)
            scale(b)
            scatter(b)
        return carry

    lax.fori_loop(0, nq, body, 0)
    wait_scatter(1)
    wait_scatter(2)
    plsc.subcore_barrier()
    pltpu.sync_copy(acc_sh.at[pl.ds(r0, _RPT)],
                    out_hbm.at[pl.ds(c * _NP + r0, _RPT)])


def _spmm_partials(src, dst, val, x_mat):
    mesh = plsc.VectorSubcoreMesh(
        core_axis_name="c", subcore_axis_name="s",
        num_cores=_NC, num_subcores=_NS)
    kern = pl.kernel(
        _spmm_body,
        out_type=jax.ShapeDtypeStruct((_NC * _NP, _D), jnp.float32),
        mesh=mesh,
        scratch_types=[
            pltpu.VMEM((_NCHUNK, _CHUNK), jnp.int32),
            [pltpu.VMEM((1, _CHUNK), jnp.int32)] * 3,
            [pltpu.VMEM((_CHUNK,), jnp.float32)] * 3,
            [pltpu.VMEM((_CHUNK, _D), jnp.float32)] * 3,
            pltpu.VMEM_SHARED((_NP, _D), jnp.float32),
            [pltpu.SemaphoreType.DMA] * 3,
            [pltpu.SemaphoreType.DMA] * 3,
            [pltpu.SemaphoreType.DMA] * 3,
        ],
    )
    return kern(src, dst, val, x_mat)


def _pad_edges(idx, val):
    npad = _EP - _E
    pad_i = (jnp.arange(npad, dtype=jnp.int32) % _N)
    dst = jnp.concatenate([idx[0], pad_i]).reshape(_NW, _NCHUNK, 1, _CHUNK)
    src = jnp.concatenate([idx[1], pad_i]).reshape(_NW, _NCHUNK, _CHUNK)
    v = jnp.concatenate([val, jnp.zeros((npad,), val.dtype)])
    return src, dst, v.reshape(_NW, _NCHUNK, _CHUNK)


def kernel(K_value, index, normed_A_value, A_index, x, n1, n2,
           W1, b1, W2, b2, W3, b3, W4, b4):
    x2d = x.reshape(_N, _D)
    x1, m2 = _mlps(x2d, W1, b1, W2, b2, W3, b3, W4, b4)

    src1, dst1, val1 = _pad_edges(index, K_value)
    p = _spmm_partials(src1, dst1, val1, x1)
    wx = _combine((p[:_N], p[_NP:_NP + _N]))

    src2, dst2, val2 = _pad_edges(A_index, normed_A_value)
    q = _spmm_partials(src2, dst2, val2, wx)
    out = _combine((q[:_N], q[_NP:_NP + _N]), extra=m2)
    return out[None]
